# baseline (speedup 1.0000x reference)
import functools

import jax
import jax.numpy as jnp
from jax import lax
from jax.experimental import pallas as pl
from jax.experimental.pallas import tpu as pltpu
from jax.experimental.pallas import tpu_sc as plsc

NC = 2
NS = 16
NW = NC * NS
CHUNK = 128

_MESH = dict(core_axis_name="c", subcore_axis_name="s")


def _ab_body(h_ref, wt_ref, wb_ref, bm1_ref, a_ref, b_ref):
    h = h_ref[...]
    a_ref[...] = (
        jnp.dot(h, wt_ref[...], preferred_element_type=jnp.float32) + bm1_ref[...]
    ).astype(jnp.bfloat16)
    b_ref[...] = jnp.dot(
        h, wb_ref[...], preferred_element_type=jnp.float32
    ).astype(jnp.bfloat16)


def _gather_body(
    a_hbm, b_hbm, row_hbm, col_hbm,
    ar_hbm, bc_hbm,
    ir0, ic0, ir1, ic1,
    bufa0, bufb0,
    bufa1, bufb1,
    isem0, isem1, gsem0, gsem1, wsem0, wsem1,
    *, cpw, epw,
):
    sid = lax.axis_index("s")
    wid = sid * NC + lax.axis_index("c")
    base = wid * epw

    slots = (
        (ir0, ic0, bufa0, bufb0, isem0, gsem0, wsem0),
        (ir1, ic1, bufa1, bufb1, isem1, gsem1, wsem1),
    )

    def issue_idx(j, s):
        ir, ic = slots[s][0], slots[s][1]
        isem = slots[s][4]
        off = base + j * CHUNK
        pltpu.async_copy(row_hbm.at[pl.ds(off, CHUNK)], ir, isem)
        pltpu.async_copy(col_hbm.at[pl.ds(off, CHUNK)], ic, isem)

    def wait_idx(s):
        ir, ic = slots[s][0], slots[s][1]
        isem = slots[s][4]
        pltpu.make_async_copy(row_hbm.at[pl.ds(0, CHUNK)], ir, isem).wait()
        pltpu.make_async_copy(col_hbm.at[pl.ds(0, CHUNK)], ic, isem).wait()

    def issue_gather(s):
        ir, ic, ba, bb, _, gs, _w = slots[s]
        pltpu.async_copy(a_hbm.at[ir], ba, gs)
        pltpu.async_copy(b_hbm.at[ic], bb, gs)

    def wait_gather(s):
        _, _, ba, bb, _, gs, _w = slots[s]
        pltpu.make_async_copy(a_hbm.at[pl.ds(0, CHUNK)], ba, gs).wait()
        pltpu.make_async_copy(b_hbm.at[pl.ds(0, CHUNK)], bb, gs).wait()

    def issue_write(j, s):
        _, _, ba, bb, _, _g, ws = slots[s]
        off = base + j * CHUNK
        pltpu.async_copy(ba, ar_hbm.at[pl.ds(off, CHUNK)], ws)
        pltpu.async_copy(bb, bc_hbm.at[pl.ds(off, CHUNK)], ws)

    def wait_write(s):
        _, _, ba, bb, _, _g, ws = slots[s]
        pltpu.make_async_copy(ba, ar_hbm.at[pl.ds(0, CHUNK)], ws).wait()
        pltpu.make_async_copy(bb, bc_hbm.at[pl.ds(0, CHUNK)], ws).wait()

    issue_idx(0, 0)
    wait_idx(0)
    issue_gather(0)
    issue_idx(1, 1)

    def half(jn, s_cur, s_next):
        @pl.when(jn >= 1)
        def _():
            wait_write(s_next)

        @pl.when(jn + 1 < cpw)
        def _():
            wait_idx(s_next)
            issue_gather(s_next)

        wait_gather(s_cur)

        @pl.when(jn + 2 < cpw)
        def _():
            issue_idx(jn + 2, s_cur)

        issue_write(jn, s_cur)

    def outer(i, carry):
        half(2 * i, 0, 1)
        half(2 * i + 1, 1, 0)
        return carry

    lax.fori_loop(0, cpw // 2, outer, 0)
    wait_write(1)


def _rel_body(
    x4f_hbm, row_hbm, col_hbm,
    rel_hbm,
    x4v,
    ir0, ic0, ir1, ic1, rb0, rb1,
    isem0, isem1, wsem0, wsem1,
    *, cpw, epw,
):
    sid = lax.axis_index("s")
    wid = sid * NC + lax.axis_index("c")
    base = wid * epw
    pltpu.sync_copy(x4f_hbm, x4v)

    slots = ((ir0, ic0, rb0, isem0, wsem0), (ir1, ic1, rb1, isem1, wsem1))

    def issue_idx(j, s):
        ir, ic, _, isem, _w = slots[s]
        off = base + j * CHUNK
        pltpu.async_copy(row_hbm.at[pl.ds(off, CHUNK)], ir, isem)
        pltpu.async_copy(col_hbm.at[pl.ds(off, CHUNK)], ic, isem)

    def wait_idx(s):
        ir, ic, _, isem, _w = slots[s]
        pltpu.make_async_copy(row_hbm.at[pl.ds(0, CHUNK)], ir, isem).wait()
        pltpu.make_async_copy(col_hbm.at[pl.ds(0, CHUNK)], ic, isem).wait()

    def compute_rel(s):
        ir, ic, rb, _, _w = slots[s]
        it = lax.iota(jnp.int32, 16)
        rep = lax.shift_right_logical(it, 2)
        lane = lax.bitwise_and(it, 3)
        for k in range(CHUNK // 4):
            rv = plsc.load_gather(ir, [k * 4 + rep])
            cv = plsc.load_gather(ic, [k * 4 + rep])
            xr = plsc.load_gather(x4v, [rv * 4 + lane])
            xc = plsc.load_gather(x4v, [cv * 4 + lane])
            rb[pl.ds(k * 16, 16)] = xr - xc

    def issue_write(j, s):
        rb, ws = slots[s][2], slots[s][4]
        off = (base + j * CHUNK) * 4
        pltpu.async_copy(rb, rel_hbm.at[pl.ds(off, CHUNK * 4)], ws)

    def wait_write(s):
        rb, ws = slots[s][2], slots[s][4]
        pltpu.make_async_copy(rb, rel_hbm.at[pl.ds(0, CHUNK * 4)], ws).wait()

    issue_idx(0, 0)

    def half(jn, s_cur, s_next):
        @pl.when(jn + 1 < cpw)
        def _():
            issue_idx(jn + 1, s_next)

        wait_idx(s_cur)

        @pl.when(jn >= 2)
        def _():
            wait_write(s_cur)

        compute_rel(s_cur)
        issue_write(jn, s_cur)

    def outer(i, carry):
        half(2 * i, 0, 1)
        half(2 * i + 1, 1, 0)
        return carry

    lax.fori_loop(0, cpw // 2, outer, 0)
    wait_write(0)
    wait_write(1)


def _edge_body(ar_ref, bc_ref, rel_ref, wd_ref, wm2_ref, bm2_ref,
               wc1_ref, bc1_ref, wc2_ref, msg_ref, c4_ref):
    rel = rel_ref[...]
    d2 = jnp.sum(rel * rel, axis=1, keepdims=True)
    dist = jnp.sqrt(d2)
    m_in = (ar_ref[...].astype(jnp.float32) + bc_ref[...].astype(jnp.float32)
            + dist * wd_ref[...])
    m = jax.nn.silu(m_in)
    msg = jax.nn.silu(
        jnp.dot(m, wm2_ref[...], preferred_element_type=jnp.float32) + bm2_ref[...]
    )
    msg_ref[...] = msg
    cm = jax.nn.silu(
        jnp.dot(msg, wc1_ref[...], preferred_element_type=jnp.float32) + bc1_ref[...]
    )
    cmult = jnp.tanh(jnp.sum(cm * wc2_ref[...], axis=1, keepdims=True))
    rel_dir = rel / (dist + 1e-8)
    coord = cmult * rel_dir
    z = jnp.zeros_like(msg[:, 4:])
    c128 = jnp.concatenate([coord, z], axis=1)
    lane = lax.broadcasted_iota(jnp.int32, c128.shape, 1)
    c4_ref[...] = jnp.where(lane == 3, 1.0, c128)


def _scatter_body(
    msg_hbm, col_hbm, zd_hbm,
    agg_hbm,
    cv0, cv1, bufm0, bufm1, aggs,
    rsem0, rsem1,
    *, cpt, np_rows,
):
    cid = lax.axis_index("c")
    sid = lax.axis_index("s")
    wid = sid * NC + cid
    rt = np_rows // NS
    pltpu.sync_copy(zd_hbm.at[pl.ds(sid * rt, rt)], aggs.at[pl.ds(sid * rt, rt)])
    plsc.subcore_barrier()

    slots = ((cv0, bufm0, rsem0), (cv1, bufm1, rsem1))

    def issue_read(j, s):
        cv, bm, rs = slots[s]
        off = (wid * cpt + j) * CHUNK
        pltpu.async_copy(msg_hbm.at[pl.ds(off, CHUNK)], bm, rs)
        pltpu.async_copy(col_hbm.at[pl.ds(off, CHUNK)], cv, rs)

    def wait_read(s):
        cv, bm, rs = slots[s]
        pltpu.make_async_copy(msg_hbm.at[pl.ds(0, CHUNK)], bm, rs).wait()
        pltpu.make_async_copy(col_hbm.at[pl.ds(0, CHUNK)], cv, rs).wait()

    issue_read(0, 0)

    def half(jn, s_cur, s_next):
        @pl.when(jn + 1 < cpt)
        def _():
            issue_read(jn + 1, s_next)

        wait_read(s_cur)
        cv, bm, _ = slots[s_cur]
        pltpu.sync_copy(bm, aggs.at[cv], add=True)

    def outer(i, carry):
        half(2 * i, 0, 1)
        half(2 * i + 1, 1, 0)
        return carry

    lax.fori_loop(0, cpt // 2, outer, 0)
    plsc.subcore_barrier()
    pltpu.sync_copy(aggs.at[pl.ds(sid * rt, rt)],
                    agg_hbm.at[pl.ds(cid * np_rows + sid * rt, rt)])


def _node_body(h_ref, agg2_ref, cu_ref, x4_ref, wn1t_ref, wn1b_ref,
               bn1_ref, wn2_ref, bn2_ref, g_ref, bt_ref, hnew_ref, xnew_ref):
    h = h_ref[...]
    agg = agg2_ref[0] + agg2_ref[1]
    t = jax.nn.silu(
        jnp.dot(h, wn1t_ref[...], preferred_element_type=jnp.float32)
        + jnp.dot(agg, wn1b_ref[...], preferred_element_type=jnp.float32)
        + bn1_ref[...]
    )
    hu = jnp.dot(t, wn2_ref[...], preferred_element_type=jnp.float32) + bn2_ref[...]
    hn = h + hu
    mean = jnp.mean(hn, axis=1, keepdims=True)
    cen = hn - mean
    var = jnp.mean(cen * cen, axis=1, keepdims=True)
    hnew_ref[...] = cen * lax.rsqrt(var + 1e-5) * g_ref[...] + bt_ref[...]
    cu = cu_ref[0][:, :4] + cu_ref[1][:, :4]
    deg = cu[:, 3:4]
    xnew_ref[...] = x4_ref[...] + cu / (deg + 1.0)


def kernel(h, x, edge_index, Wm1, bm1, Wm2, bm2, Wc1, bc1, Wc2,
           Wn1, bn1, Wn2, bn2, gamma, beta):
    n, d = h.shape
    e = edge_index.shape[1]
    hw = d // 2
    f32 = jnp.float32

    npad = ((n + 1023) // 1024) * 1024
    cpw = -(-e // (NW * CHUNK))
    cpw += cpw % 2
    ep = NW * CHUNK * cpw
    epw = cpw * CHUNK
    g_chunks = ep // CHUNK
    cpt = g_chunks // NS

    row = edge_index[0].astype(jnp.int32)
    col = edge_index[1].astype(jnp.int32)
    pad = ep - e
    pad_idx = n + jnp.arange(pad, dtype=jnp.int32) % (npad - n)
    row_p = jnp.concatenate([row, pad_idx])
    col_p = jnp.concatenate([col, pad_idx])

    h_p = jnp.pad(h, ((0, npad - n), (0, 0)))
    x4 = jnp.pad(x, ((0, 0), (0, 1)))
    x4_p = jnp.pad(x4, ((0, npad - n), (0, 0)))

    wm1t = Wm1[:d]
    wm1b = Wm1[d:2 * d]
    wd = Wm1[2 * d].reshape(1, d)
    bm1r = bm1.reshape(1, d)
    bm2r = bm2.reshape(1, d)
    bc1r = bc1.reshape(1, d)
    wc2r = Wc2[:, 0].reshape(1, d)
    wn1t = Wn1[:d]
    wn1bl = Wn1[d:d + hw]
    wn1br = Wn1[d + hw:]
    bn1r = bn1.reshape(1, d)
    bn2r = bn2.reshape(1, d)
    gr = gamma.reshape(1, d)
    br = beta.reshape(1, d)

    nb = npad // 1024
    full = lambda s: pl.BlockSpec(s, lambda i: (0,) * len(s))
    a_tab, b_tab = pl.pallas_call(
        _ab_body,
        grid=(nb,),
        in_specs=[
            pl.BlockSpec((1024, d), lambda i: (i, 0)),
            full((d, d)), full((d, d)), full((1, d)),
        ],
        out_specs=[
            pl.BlockSpec((1024, d), lambda i: (i, 0)),
            pl.BlockSpec((1024, d), lambda i: (i, 0)),
        ],
        out_shape=[
            jax.ShapeDtypeStruct((npad, d), jnp.bfloat16),
            jax.ShapeDtypeStruct((npad, d), jnp.bfloat16),
        ],
    )(h_p, wm1t, wm1b, bm1r)

    mesh = plsc.VectorSubcoreMesh(**_MESH)
    gather_fn = pl.kernel(
        functools.partial(_gather_body, cpw=cpw, epw=epw),
        out_type=[
            jax.ShapeDtypeStruct((ep, d), jnp.bfloat16),
            jax.ShapeDtypeStruct((ep, d), jnp.bfloat16),
        ],
        mesh=mesh,
        compiler_params=pltpu.CompilerParams(use_tc_tiling_on_sc=False),
        scratch_types=[
            pltpu.VMEM((CHUNK,), jnp.int32), pltpu.VMEM((CHUNK,), jnp.int32),
            pltpu.VMEM((CHUNK,), jnp.int32), pltpu.VMEM((CHUNK,), jnp.int32),
            pltpu.VMEM((CHUNK, d), jnp.bfloat16), pltpu.VMEM((CHUNK, d), jnp.bfloat16),
            pltpu.VMEM((CHUNK, d), jnp.bfloat16), pltpu.VMEM((CHUNK, d), jnp.bfloat16),
            pltpu.SemaphoreType.DMA, pltpu.SemaphoreType.DMA,
            pltpu.SemaphoreType.DMA, pltpu.SemaphoreType.DMA,
            pltpu.SemaphoreType.DMA, pltpu.SemaphoreType.DMA,
        ],
    )
    ar, bc_g = gather_fn(a_tab, b_tab, row_p, col_p)

    rel_fn = pl.kernel(
        functools.partial(_rel_body, cpw=cpw, epw=epw),
        out_type=[jax.ShapeDtypeStruct((ep * 4,), f32)],
        mesh=mesh,
        compiler_params=pltpu.CompilerParams(
            use_tc_tiling_on_sc=False, needs_layout_passes=False),
        scratch_types=[
            pltpu.VMEM((npad * 4,), f32),
            pltpu.VMEM((CHUNK,), jnp.int32), pltpu.VMEM((CHUNK,), jnp.int32),
            pltpu.VMEM((CHUNK,), jnp.int32), pltpu.VMEM((CHUNK,), jnp.int32),
            pltpu.VMEM((CHUNK * 4,), f32), pltpu.VMEM((CHUNK * 4,), f32),
            pltpu.SemaphoreType.DMA, pltpu.SemaphoreType.DMA,
            pltpu.SemaphoreType.DMA, pltpu.SemaphoreType.DMA,
        ],
    )
    rel4 = rel_fn(x4_p.reshape(-1), row_p, col_p)[0].reshape(ep, 4)

    eb = 2048
    msg, c4 = pl.pallas_call(
        _edge_body,
        grid=(ep // eb,),
        in_specs=[
            pl.BlockSpec((eb, d), lambda i: (i, 0)),
            pl.BlockSpec((eb, d), lambda i: (i, 0)),
            pl.BlockSpec((eb, 4), lambda i: (i, 0)),
            full((1, d)), full((d, d)), full((1, d)),
            full((d, d)), full((1, d)), full((1, d)),
        ],
        out_specs=[
            pl.BlockSpec((eb, d), lambda i: (i, 0)),
            pl.BlockSpec((eb, d), lambda i: (i, 0)),
        ],
        out_shape=[
            jax.ShapeDtypeStruct((ep, d), f32),
            jax.ShapeDtypeStruct((ep, d), f32),
        ],
    )(ar, bc_g, rel4, wd, Wm2, bm2r, Wc1, bc1r, wc2r)

    cpt = g_chunks // NW
    zd = jnp.zeros((npad, d), f32)
    scatter_fn = pl.kernel(
        functools.partial(_scatter_body, cpt=cpt, np_rows=npad),
        out_type=[jax.ShapeDtypeStruct((NC * npad, d), f32)],
        mesh=mesh,
        scratch_types=[
            pltpu.VMEM((CHUNK,), jnp.int32), pltpu.VMEM((CHUNK,), jnp.int32),
            pltpu.VMEM((CHUNK, d), f32), pltpu.VMEM((CHUNK, d), f32),
            pltpu.VMEM_SHARED((npad, d), f32),
            pltpu.SemaphoreType.DMA, pltpu.SemaphoreType.DMA,
        ],
    )
    agg2 = scatter_fn(msg, col_p, zd)[0].reshape(NC, npad, d)
    cu2 = scatter_fn(c4, col_p, zd)[0].reshape(NC, npad, d)

    nbn = 10
    bnr = n // nbn
    hnew, xnew4 = pl.pallas_call(
        _node_body,
        grid=(nbn,),
        in_specs=[
            pl.BlockSpec((bnr, d), lambda i: (i, 0)),
            pl.BlockSpec((NC, bnr, d), lambda i: (0, i, 0)),
            pl.BlockSpec((NC, bnr, d), lambda i: (0, i, 0)),
            pl.BlockSpec((bnr, 4), lambda i: (i, 0)),
            full((d, d)), full((d, d)), full((1, d)),
            full((d, d)), full((1, d)), full((1, d)), full((1, d)),
        ],
        out_specs=[
            pl.BlockSpec((bnr, d), lambda i: (i, 0)),
            pl.BlockSpec((bnr, 4), lambda i: (i, 0)),
        ],
        out_shape=[
            jax.ShapeDtypeStruct((n, d), f32),
            jax.ShapeDtypeStruct((n, 4), f32),
        ],
    )(h, agg2, cu2, x4, wn1t, Wn1[d:], bn1r, Wn2, bn2r, gr, br)

    return hnew, xnew4[:, :3]


# speedup vs baseline: 1.6465x; 1.6465x over previous
import functools

import jax
import jax.numpy as jnp
from jax import lax
from jax.experimental import pallas as pl
from jax.experimental.pallas import tpu as pltpu
from jax.experimental.pallas import tpu_sc as plsc

NC = 2
NS = 16
NW = NC * NS
CHUNK = 128

_MESH = dict(core_axis_name="c", subcore_axis_name="s")


def _ab_body(h_ref, wt_ref, wb_ref, bm1_ref, a_ref, b_ref):
    h = h_ref[...]
    a_ref[...] = (
        jnp.dot(h, wt_ref[...], preferred_element_type=jnp.float32) + bm1_ref[...]
    )
    b_ref[...] = jnp.dot(h, wb_ref[...], preferred_element_type=jnp.float32)


def _gather_body(
    a_hbm, b_hbm, row_hbm, col_hbm,
    ar_hbm, bc_hbm,
    ir0, ic0, ir1, ic1,
    bufa0, bufb0,
    bufa1, bufb1,
    isem0, isem1, gsem0, gsem1, wsem0, wsem1,
    *, cpw, epw,
):
    sid = lax.axis_index("s")
    wid = sid * NC + lax.axis_index("c")
    base = wid * epw

    slots = (
        (ir0, ic0, bufa0, bufb0, isem0, gsem0, wsem0),
        (ir1, ic1, bufa1, bufb1, isem1, gsem1, wsem1),
    )

    def issue_idx(j, s):
        ir, ic = slots[s][0], slots[s][1]
        isem = slots[s][4]
        off = base + j * CHUNK
        pltpu.async_copy(row_hbm.at[pl.ds(off, CHUNK)], ir, isem)
        pltpu.async_copy(col_hbm.at[pl.ds(off, CHUNK)], ic, isem)

    def wait_idx(s):
        ir, ic = slots[s][0], slots[s][1]
        isem = slots[s][4]
        pltpu.make_async_copy(row_hbm.at[pl.ds(0, CHUNK)], ir, isem).wait()
        pltpu.make_async_copy(col_hbm.at[pl.ds(0, CHUNK)], ic, isem).wait()

    def issue_gather(s):
        ir, ic, ba, bb, _, gs, _w = slots[s]
        pltpu.async_copy(a_hbm.at[ir], ba, gs)
        pltpu.async_copy(b_hbm.at[ic], bb, gs)

    def wait_gather(s):
        _, _, ba, bb, _, gs, _w = slots[s]
        pltpu.make_async_copy(a_hbm.at[pl.ds(0, CHUNK)], ba, gs).wait()
        pltpu.make_async_copy(b_hbm.at[pl.ds(0, CHUNK)], bb, gs).wait()

    def issue_write(j, s):
        _, _, ba, bb, _, _g, ws = slots[s]
        off = base + j * CHUNK
        pltpu.async_copy(ba, ar_hbm.at[pl.ds(off, CHUNK)], ws)
        pltpu.async_copy(bb, bc_hbm.at[pl.ds(off, CHUNK)], ws)

    def wait_write(s):
        _, _, ba, bb, _, _g, ws = slots[s]
        pltpu.make_async_copy(ba, ar_hbm.at[pl.ds(0, CHUNK)], ws).wait()
        pltpu.make_async_copy(bb, bc_hbm.at[pl.ds(0, CHUNK)], ws).wait()

    issue_idx(0, 0)
    wait_idx(0)
    issue_gather(0)
    issue_idx(1, 1)

    def half(jn, s_cur, s_next):
        @pl.when(jn >= 1)
        def _():
            wait_write(s_next)

        @pl.when(jn + 1 < cpw)
        def _():
            wait_idx(s_next)
            issue_gather(s_next)

        wait_gather(s_cur)

        @pl.when(jn + 2 < cpw)
        def _():
            issue_idx(jn + 2, s_cur)

        issue_write(jn, s_cur)

    def outer(i, carry):
        half(2 * i, 0, 1)
        half(2 * i + 1, 1, 0)
        return carry

    lax.fori_loop(0, cpw // 2, outer, 0)
    wait_write(1)


def _rel_body(
    x4f_hbm, row_hbm, col_hbm,
    rel_hbm,
    x4v,
    ir0, ic0, ir1, ic1, rb0, rb1,
    isem0, isem1, wsem0, wsem1,
    *, cpw, epw,
):
    sid = lax.axis_index("s")
    wid = sid * NC + lax.axis_index("c")
    base = wid * epw
    pltpu.sync_copy(x4f_hbm, x4v)

    slots = ((ir0, ic0, rb0, isem0, wsem0), (ir1, ic1, rb1, isem1, wsem1))

    def issue_idx(j, s):
        ir, ic, _, isem, _w = slots[s]
        off = base + j * CHUNK
        pltpu.async_copy(row_hbm.at[pl.ds(off, CHUNK)], ir, isem)
        pltpu.async_copy(col_hbm.at[pl.ds(off, CHUNK)], ic, isem)

    def wait_idx(s):
        ir, ic, _, isem, _w = slots[s]
        pltpu.make_async_copy(row_hbm.at[pl.ds(0, CHUNK)], ir, isem).wait()
        pltpu.make_async_copy(col_hbm.at[pl.ds(0, CHUNK)], ic, isem).wait()

    def compute_rel(s):
        ir, ic, rb, _, _w = slots[s]
        it = lax.iota(jnp.int32, 16)
        rep = lax.shift_right_logical(it, 2)
        lane = lax.bitwise_and(it, 3)
        for k in range(CHUNK // 4):
            rv = plsc.load_gather(ir, [k * 4 + rep])
            cv = plsc.load_gather(ic, [k * 4 + rep])
            xr = plsc.load_gather(x4v, [rv * 4 + lane])
            xc = plsc.load_gather(x4v, [cv * 4 + lane])
            rb[pl.ds(k * 16, 16)] = xr - xc

    def issue_write(j, s):
        rb, ws = slots[s][2], slots[s][4]
        off = (base + j * CHUNK) * 4
        pltpu.async_copy(rb, rel_hbm.at[pl.ds(off, CHUNK * 4)], ws)

    def wait_write(s):
        rb, ws = slots[s][2], slots[s][4]
        pltpu.make_async_copy(rb, rel_hbm.at[pl.ds(0, CHUNK * 4)], ws).wait()

    issue_idx(0, 0)

    def half(jn, s_cur, s_next):
        @pl.when(jn + 1 < cpw)
        def _():
            issue_idx(jn + 1, s_next)

        wait_idx(s_cur)

        @pl.when(jn >= 2)
        def _():
            wait_write(s_cur)

        compute_rel(s_cur)
        issue_write(jn, s_cur)

    def outer(i, carry):
        half(2 * i, 0, 1)
        half(2 * i + 1, 1, 0)
        return carry

    lax.fori_loop(0, cpw // 2, outer, 0)
    wait_write(0)
    wait_write(1)


def _edge_body(ar_ref, bc_ref, rel_ref, wd_ref, wm2_ref, bm2_ref,
               wc1_ref, bc1_ref, wc2_ref, msg_ref, c4_ref):
    rel = rel_ref[...]
    d2 = jnp.sum(rel * rel, axis=1, keepdims=True)
    dist = jnp.sqrt(d2)
    m_in = ar_ref[...] + bc_ref[...] + dist * wd_ref[...]
    m = jax.nn.silu(m_in)
    msg = jax.nn.silu(
        jnp.dot(m, wm2_ref[...], preferred_element_type=jnp.float32) + bm2_ref[...]
    )
    msg_ref[...] = msg
    cm = jax.nn.silu(
        jnp.dot(msg, wc1_ref[...], preferred_element_type=jnp.float32) + bc1_ref[...]
    )
    cmult = jnp.tanh(jnp.sum(cm * wc2_ref[...], axis=1, keepdims=True))
    rel_dir = rel / (dist + 1e-8)
    coord = cmult * rel_dir
    z = jnp.zeros_like(msg[:, 4:])
    c128 = jnp.concatenate([coord, z], axis=1)
    lane = lax.broadcasted_iota(jnp.int32, c128.shape, 1)
    c4_ref[...] = jnp.where(lane == 3, 1.0, c128)


def _scatter_body(
    msg_hbm, col_hbm, zd_hbm,
    agg_hbm,
    cv0, cv1, bufm0, bufm1, aggs,
    rsem0, rsem1,
    *, cpt, np_rows,
):
    cid = lax.axis_index("c")
    sid = lax.axis_index("s")
    wid = sid * NC + cid
    rt = np_rows // NS
    pltpu.sync_copy(zd_hbm.at[pl.ds(sid * rt, rt)], aggs.at[pl.ds(sid * rt, rt)])
    plsc.subcore_barrier()

    slots = ((cv0, bufm0, rsem0), (cv1, bufm1, rsem1))

    def issue_read(j, s):
        cv, bm, rs = slots[s]
        off = (wid * cpt + j) * CHUNK
        pltpu.async_copy(msg_hbm.at[pl.ds(off, CHUNK)], bm, rs)
        pltpu.async_copy(col_hbm.at[pl.ds(off, CHUNK)], cv, rs)

    def wait_read(s):
        cv, bm, rs = slots[s]
        pltpu.make_async_copy(msg_hbm.at[pl.ds(0, CHUNK)], bm, rs).wait()
        pltpu.make_async_copy(col_hbm.at[pl.ds(0, CHUNK)], cv, rs).wait()

    issue_read(0, 0)

    def half(jn, s_cur, s_next):
        @pl.when(jn + 1 < cpt)
        def _():
            issue_read(jn + 1, s_next)

        wait_read(s_cur)
        cv, bm, _ = slots[s_cur]
        pltpu.sync_copy(bm, aggs.at[cv], add=True)

    def outer(i, carry):
        half(2 * i, 0, 1)
        half(2 * i + 1, 1, 0)
        return carry

    lax.fori_loop(0, cpt // 2, outer, 0)
    plsc.subcore_barrier()
    pltpu.sync_copy(aggs.at[pl.ds(sid * rt, rt)],
                    agg_hbm.at[pl.ds(cid * np_rows + sid * rt, rt)])


def _node_body(h_ref, agg2_ref, cu_ref, x4_ref, wn1t_ref, wn1b_ref,
               bn1_ref, wn2_ref, bn2_ref, g_ref, bt_ref, hnew_ref, xnew_ref):
    h = h_ref[...]
    agg = agg2_ref[0] + agg2_ref[1]
    t = jax.nn.silu(
        jnp.dot(h, wn1t_ref[...], preferred_element_type=jnp.float32)
        + jnp.dot(agg, wn1b_ref[...], preferred_element_type=jnp.float32)
        + bn1_ref[...]
    )
    hu = jnp.dot(t, wn2_ref[...], preferred_element_type=jnp.float32) + bn2_ref[...]
    hn = h + hu
    mean = jnp.mean(hn, axis=1, keepdims=True)
    cen = hn - mean
    var = jnp.mean(cen * cen, axis=1, keepdims=True)
    hnew_ref[...] = cen * lax.rsqrt(var + 1e-5) * g_ref[...] + bt_ref[...]
    cu = cu_ref[0][:, :4] + cu_ref[1][:, :4]
    deg = cu[:, 3:4]
    xnew_ref[...] = x4_ref[...] + cu / (deg + 1.0)


def kernel(h, x, edge_index, Wm1, bm1, Wm2, bm2, Wc1, bc1, Wc2,
           Wn1, bn1, Wn2, bn2, gamma, beta):
    n, d = h.shape
    e = edge_index.shape[1]
    hw = d // 2
    f32 = jnp.float32

    npad = ((n + 1023) // 1024) * 1024
    cpw = -(-e // (NW * CHUNK))
    cpw += cpw % 2
    ep = NW * CHUNK * cpw
    epw = cpw * CHUNK
    g_chunks = ep // CHUNK
    cpt = g_chunks // NS

    row = edge_index[0].astype(jnp.int32)
    col = edge_index[1].astype(jnp.int32)
    pad = ep - e
    pad_idx = n + jnp.arange(pad, dtype=jnp.int32) % (npad - n)
    row_p = jnp.concatenate([row, pad_idx])
    col_p = jnp.concatenate([col, pad_idx])

    h_p = jnp.pad(h, ((0, npad - n), (0, 0)))
    x4 = jnp.pad(x, ((0, 0), (0, 1)))
    x4_p = jnp.pad(x4, ((0, npad - n), (0, 0)))

    wm1t = Wm1[:d]
    wm1b = Wm1[d:2 * d]
    wd = Wm1[2 * d].reshape(1, d)
    bm1r = bm1.reshape(1, d)
    bm2r = bm2.reshape(1, d)
    bc1r = bc1.reshape(1, d)
    wc2r = Wc2[:, 0].reshape(1, d)
    wn1t = Wn1[:d]
    wn1bl = Wn1[d:d + hw]
    wn1br = Wn1[d + hw:]
    bn1r = bn1.reshape(1, d)
    bn2r = bn2.reshape(1, d)
    gr = gamma.reshape(1, d)
    br = beta.reshape(1, d)

    nb = npad // 1024
    full = lambda s: pl.BlockSpec(s, lambda i: (0,) * len(s))
    a_tab, b_tab = pl.pallas_call(
        _ab_body,
        grid=(nb,),
        in_specs=[
            pl.BlockSpec((1024, d), lambda i: (i, 0)),
            full((d, d)), full((d, d)), full((1, d)),
        ],
        out_specs=[
            pl.BlockSpec((1024, d), lambda i: (i, 0)),
            pl.BlockSpec((1024, d), lambda i: (i, 0)),
        ],
        out_shape=[
            jax.ShapeDtypeStruct((npad, d), f32),
            jax.ShapeDtypeStruct((npad, d), f32),
        ],
    )(h_p, wm1t, wm1b, bm1r)

    mesh = plsc.VectorSubcoreMesh(**_MESH)
    gather_fn = pl.kernel(
        functools.partial(_gather_body, cpw=cpw, epw=epw),
        out_type=[
            jax.ShapeDtypeStruct((ep, d), f32),
            jax.ShapeDtypeStruct((ep, d), f32),
        ],
        mesh=mesh,
        compiler_params=pltpu.CompilerParams(use_tc_tiling_on_sc=False),
        scratch_types=[
            pltpu.VMEM((CHUNK,), jnp.int32), pltpu.VMEM((CHUNK,), jnp.int32),
            pltpu.VMEM((CHUNK,), jnp.int32), pltpu.VMEM((CHUNK,), jnp.int32),
            pltpu.VMEM((CHUNK, d), f32), pltpu.VMEM((CHUNK, d), f32),
            pltpu.VMEM((CHUNK, d), f32), pltpu.VMEM((CHUNK, d), f32),
            pltpu.SemaphoreType.DMA, pltpu.SemaphoreType.DMA,
            pltpu.SemaphoreType.DMA, pltpu.SemaphoreType.DMA,
            pltpu.SemaphoreType.DMA, pltpu.SemaphoreType.DMA,
        ],
    )
    ar, bc_g = gather_fn(a_tab, b_tab, row_p, col_p)

    rel_fn = pl.kernel(
        functools.partial(_rel_body, cpw=cpw, epw=epw),
        out_type=[jax.ShapeDtypeStruct((ep * 4,), f32)],
        mesh=mesh,
        compiler_params=pltpu.CompilerParams(
            use_tc_tiling_on_sc=False, needs_layout_passes=False),
        scratch_types=[
            pltpu.VMEM((npad * 4,), f32),
            pltpu.VMEM((CHUNK,), jnp.int32), pltpu.VMEM((CHUNK,), jnp.int32),
            pltpu.VMEM((CHUNK,), jnp.int32), pltpu.VMEM((CHUNK,), jnp.int32),
            pltpu.VMEM((CHUNK * 4,), f32), pltpu.VMEM((CHUNK * 4,), f32),
            pltpu.SemaphoreType.DMA, pltpu.SemaphoreType.DMA,
            pltpu.SemaphoreType.DMA, pltpu.SemaphoreType.DMA,
        ],
    )
    rel4 = rel_fn(x4_p.reshape(-1), row_p, col_p)[0].reshape(ep, 4)

    eb = 2048
    msg, c4 = pl.pallas_call(
        _edge_body,
        grid=(ep // eb,),
        in_specs=[
            pl.BlockSpec((eb, d), lambda i: (i, 0)),
            pl.BlockSpec((eb, d), lambda i: (i, 0)),
            pl.BlockSpec((eb, 4), lambda i: (i, 0)),
            full((1, d)), full((d, d)), full((1, d)),
            full((d, d)), full((1, d)), full((1, d)),
        ],
        out_specs=[
            pl.BlockSpec((eb, d), lambda i: (i, 0)),
            pl.BlockSpec((eb, d), lambda i: (i, 0)),
        ],
        out_shape=[
            jax.ShapeDtypeStruct((ep, d), f32),
            jax.ShapeDtypeStruct((ep, d), f32),
        ],
    )(ar, bc_g, rel4, wd, Wm2, bm2r, Wc1, bc1r, wc2r)

    cpt = g_chunks // NW
    zd = jnp.zeros((npad, d), f32)
    scatter_fn = pl.kernel(
        functools.partial(_scatter_body, cpt=cpt, np_rows=npad),
        out_type=[jax.ShapeDtypeStruct((NC * npad, d), f32)],
        mesh=mesh,
        scratch_types=[
            pltpu.VMEM((CHUNK,), jnp.int32), pltpu.VMEM((CHUNK,), jnp.int32),
            pltpu.VMEM((CHUNK, d), f32), pltpu.VMEM((CHUNK, d), f32),
            pltpu.VMEM_SHARED((npad, d), f32),
            pltpu.SemaphoreType.DMA, pltpu.SemaphoreType.DMA,
        ],
    )
    agg2 = scatter_fn(msg, col_p, zd)[0].reshape(NC, npad, d)
    cu2 = scatter_fn(c4, col_p, zd)[0].reshape(NC, npad, d)

    nbn = 10
    bnr = n // nbn
    hnew, xnew4 = pl.pallas_call(
        _node_body,
        grid=(nbn,),
        in_specs=[
            pl.BlockSpec((bnr, d), lambda i: (i, 0)),
            pl.BlockSpec((NC, bnr, d), lambda i: (0, i, 0)),
            pl.BlockSpec((NC, bnr, d), lambda i: (0, i, 0)),
            pl.BlockSpec((bnr, 4), lambda i: (i, 0)),
            full((d, d)), full((d, d)), full((1, d)),
            full((d, d)), full((1, d)), full((1, d)), full((1, d)),
        ],
        out_specs=[
            pl.BlockSpec((bnr, d), lambda i: (i, 0)),
            pl.BlockSpec((bnr, 4), lambda i: (i, 0)),
        ],
        out_shape=[
            jax.ShapeDtypeStruct((n, d), f32),
            jax.ShapeDtypeStruct((n, 4), f32),
        ],
    )(h, agg2, cu2, x4, wn1t, Wn1[d:], bn1r, Wn2, bn2r, gr, br)

    return hnew, xnew4[:, :3]
